# baseline (speedup 1.0000x reference)
import functools

import jax
import jax.numpy as jnp
from jax import lax
from jax.experimental import pallas as pl
from jax.experimental.pallas import tpu as pltpu
from jax.experimental.pallas import tpu_sc as plsc

N = 100000
D = 128
G = 1024
DG = 64
DOUT = 2 * D + DG
CHUNK = 128
NFULL = N // CHUNK
TAIL = N - NFULL * CHUNK
NSUB = 16
ITERS = -(-NFULL // NSUB)
GROWS = G // NSUB


def _segment_sum_body(sid, feats, ids, out, col0, acc,
                      rows_a, rows_b, idx_a, idx_b, sem_a, sem_b,
                      rows_t, idx_t):
    slot_a = (rows_a, idx_a, sem_a)
    slot_b = (rows_b, idx_b, sem_b)

    def issue(slot, c):
        rows, idx, sem = slot
        off = c * CHUNK
        pltpu.async_copy(feats.at[pl.ds(off, CHUNK)], rows, sem)
        pltpu.async_copy(ids.at[pl.ds(off, CHUNK)], idx, sem)

    def drain(slot, c):
        rows, idx, sem = slot
        off = c * CHUNK
        pltpu.make_async_copy(feats.at[pl.ds(off, CHUNK)], rows, sem).wait()
        pltpu.make_async_copy(ids.at[pl.ds(off, CHUNK)], idx, sem).wait()

    issue(slot_a, sid)

    def zero_row(r, _):
        for j in range(D // 16):
            rows_t[r, pl.ds(j * 16, 16)] = jnp.zeros((16,), jnp.float32)
        return _

    lax.fori_loop(0, TAIL, zero_row, None)
    for r in range(0, GROWS, TAIL):
        pltpu.sync_copy(rows_t, acc.at[pl.ds(sid * GROWS + r, TAIL)])
    plsc.subcore_barrier()

    def step(i, cur, nxt):
        c = sid + i * NSUB
        cn = c + NSUB

        @pl.when(c < NFULL)
        def _():
            drain(cur, c)

        @pl.when(cn < NFULL)
        def _():
            issue(nxt, cn)

        @pl.when(c < NFULL)
        def _():
            pltpu.sync_copy(cur[0], acc.at[cur[1]], add=True)

    def body2(i2, _):
        step(i2 * 2, slot_a, slot_b)
        step(i2 * 2 + 1, slot_b, slot_a)
        return _

    lax.fori_loop(0, (ITERS + 1) // 2, body2, None)

    @pl.when(sid == NSUB - 1)
    def _():
        off = NFULL * CHUNK
        pltpu.sync_copy(feats.at[pl.ds(off, TAIL)], rows_t)
        pltpu.sync_copy(ids.at[pl.ds(off, TAIL)], idx_t)
        pltpu.sync_copy(rows_t, acc.at[idx_t], add=True)

    plsc.subcore_barrier()
    pltpu.sync_copy(acc.at[pl.ds(sid * GROWS, GROWS)],
                    out.at[pl.ds(sid * GROWS, GROWS), pl.ds(col0, D)])


@functools.partial(
    pl.kernel,
    out_type=jax.ShapeDtypeStruct((G, DOUT), jnp.float32),
    mesh=plsc.VectorSubcoreMesh(
        core_axis_name="c", subcore_axis_name="s", num_cores=2, num_subcores=NSUB
    ),
    scratch_types=(
        pltpu.VMEM_SHARED((G, D), jnp.float32),
        pltpu.VMEM((CHUNK, D), jnp.float32),
        pltpu.VMEM((CHUNK, D), jnp.float32),
        pltpu.VMEM((CHUNK,), jnp.int32),
        pltpu.VMEM((CHUNK,), jnp.int32),
        pltpu.SemaphoreType.DMA,
        pltpu.SemaphoreType.DMA,
        pltpu.VMEM((TAIL, D), jnp.float32),
        pltpu.VMEM((TAIL,), jnp.int32),
        pltpu.VMEM((GROWS, DG), jnp.float32),
    ),
)
def _pooled(atom_hbm, aids_hbm, bond_hbm, bids_hbm, glob_hbm, out,
            acc, rows_a, rows_b, idx_a, idx_b, sem_a, sem_b, rows_t, idx_t,
            gstage):
    cid = lax.axis_index("c")
    sid = lax.axis_index("s")

    @pl.when(cid == 0)
    def _():
        _segment_sum_body(sid, atom_hbm, aids_hbm, out, 0, acc,
                          rows_a, rows_b, idx_a, idx_b, sem_a, sem_b,
                          rows_t, idx_t)
        pltpu.sync_copy(glob_hbm.at[pl.ds(sid * GROWS, GROWS)], gstage)
        pltpu.sync_copy(gstage,
                        out.at[pl.ds(sid * GROWS, GROWS), pl.ds(2 * D, DG)])

    @pl.when(cid == 1)
    def _():
        _segment_sum_body(sid, bond_hbm, bids_hbm, out, D, acc,
                          rows_a, rows_b, idx_a, idx_b, sem_a, sem_b,
                          rows_t, idx_t)


def kernel(atom_feats, bond_feats, global_feats, atom_segment_ids, bond_segment_ids):
    return _pooled(atom_feats, atom_segment_ids, bond_feats, bond_segment_ids,
                   global_feats)
